# baseline (speedup 1.0000x reference)
import functools

import jax
import jax.numpy as jnp
from jax.experimental import pallas as pl
from jax.experimental.pallas import tpu as pltpu


def _adapter_body(x_ref, dw_ref, db_ref, uw_ref, o_ref):
    x = x_ref[0]
    dw = dw_ref[0, 0]
    db = db_ref[0, 0, 0]
    uw = uw_ref[0, 0]
    xb = x.astype(jnp.bfloat16)
    dwb = dw.astype(jnp.bfloat16)
    uwb = uw.astype(jnp.bfloat16)
    z = jnp.dot(xb, dwb, preferred_element_type=jnp.float32) + db[None, :]
    z = z * jax.nn.sigmoid(z)
    o_ref[0, 0] = jnp.dot(
        z.astype(jnp.bfloat16), uwb, preferred_element_type=jnp.float32
    )


@jax.jit
def kernel(x, expert_index, down_w, down_b, up_w):
    B, S, C = x.shape
    M, N, _, D = down_w.shape
    TS = 1024
    s_blocks = S // TS

    idx = expert_index.astype(jnp.int32)
    m = jnp.arange(M)[:, None]
    bdw = down_w[m, idx]
    bdb = down_b[m, idx].reshape(M, B, 1, D)
    buw = up_w[m, idx]

    grid = (M, B, s_blocks)

    out = pl.pallas_call(
        _adapter_body,
        grid=grid,
        in_specs=[
            pl.BlockSpec((1, TS, C), lambda mm, b, s: (b, s, 0)),
            pl.BlockSpec((1, 1, C, D), lambda mm, b, s: (mm, b, 0, 0)),
            pl.BlockSpec((1, 1, 1, D), lambda mm, b, s: (mm, b, 0, 0)),
            pl.BlockSpec((1, 1, D, C), lambda mm, b, s: (mm, b, 0, 0)),
        ],
        out_specs=pl.BlockSpec((1, 1, TS, C), lambda mm, b, s: (mm, b, s, 0)),
        out_shape=jax.ShapeDtypeStruct((M, B, S, C), jnp.float32),
        compiler_params=pltpu.CompilerParams(
            dimension_semantics=("parallel", "parallel", "parallel"),
        ),
    )(x, bdw, bdb, buw)
    return out
